# baseline (speedup 1.0000x reference)
import jax
import jax.numpy as jnp
from jax.experimental import pallas as pl

_N = 10000
_E = 160000
_EF = 16
_NF0 = 16
_NF1 = 32
_GATD = 128
_TE = 2000
_TN = 2000


def _edge_kernel(ea_ref, xs_ref, W1_ref, b1_ref, W2_ref, b2_ref, msg_ref):
    h1 = jnp.maximum(
        jnp.dot(ea_ref[...], W1_ref[...], preferred_element_type=jnp.float32)
        + b1_ref[...], 0.0)
    ew = jnp.dot(h1, W2_ref[...], preferred_element_type=jnp.float32) + b2_ref[...]
    xs = xs_ref[...]
    acc = xs[:, 0:1] * ew[:, 0:_NF1]
    for i in range(1, _NF0):
        acc = acc + xs[:, i:i + 1] * ew[:, i * _NF1:(i + 1) * _NF1]
    msg_ref[...] = acc


def _node_kernel(agg_ref, x_ref, rW_ref, rb_ref, gW_ref, asrc_ref, adst_ref,
                 h_ref, as_ref, ad_ref):
    x1 = jnp.maximum(
        agg_ref[...]
        + jnp.dot(x_ref[...], rW_ref[...], preferred_element_type=jnp.float32)
        + rb_ref[...], 0.0)
    h = jnp.dot(x1, gW_ref[...], preferred_element_type=jnp.float32)
    h_ref[...] = h
    as_ref[...] = jnp.sum(h * asrc_ref[...], axis=-1, keepdims=True)
    ad_ref[...] = jnp.sum(h * adst_ref[...], axis=-1, keepdims=True)


def kernel(x, edge_index, edge_attr, batch, W1, b1, W2, b2, root_W, root_b,
           gat_W, att_src, att_dst, gat_b, fc_W, fc_b):
    n = x.shape[0]
    src = edge_index[0]
    dst = edge_index[1]
    x_src = x[src]

    msg = pl.pallas_call(
        _edge_kernel,
        grid=(_E // _TE,),
        in_specs=[
            pl.BlockSpec((_TE, _EF), lambda i: (i, 0)),
            pl.BlockSpec((_TE, _NF0), lambda i: (i, 0)),
            pl.BlockSpec((_EF, 32), lambda i: (0, 0)),
            pl.BlockSpec((1, 32), lambda i: (0, 0)),
            pl.BlockSpec((32, _NF0 * _NF1), lambda i: (0, 0)),
            pl.BlockSpec((1, _NF0 * _NF1), lambda i: (0, 0)),
        ],
        out_specs=pl.BlockSpec((_TE, _NF1), lambda i: (i, 0)),
        out_shape=jax.ShapeDtypeStruct((_E, _NF1), jnp.float32),
    )(edge_attr, x_src, W1, b1.reshape(1, -1), W2, b2.reshape(1, -1))

    agg = jax.ops.segment_sum(msg, dst, num_segments=n)

    h, a_s, a_d = pl.pallas_call(
        _node_kernel,
        grid=(_N // _TN,),
        in_specs=[
            pl.BlockSpec((_TN, _NF1), lambda i: (i, 0)),
            pl.BlockSpec((_TN, _NF0), lambda i: (i, 0)),
            pl.BlockSpec((_NF0, _NF1), lambda i: (0, 0)),
            pl.BlockSpec((1, _NF1), lambda i: (0, 0)),
            pl.BlockSpec((_NF1, _GATD), lambda i: (0, 0)),
            pl.BlockSpec((1, _GATD), lambda i: (0, 0)),
            pl.BlockSpec((1, _GATD), lambda i: (0, 0)),
        ],
        out_specs=[
            pl.BlockSpec((_TN, _GATD), lambda i: (i, 0)),
            pl.BlockSpec((_TN, 1), lambda i: (i, 0)),
            pl.BlockSpec((_TN, 1), lambda i: (i, 0)),
        ],
        out_shape=[
            jax.ShapeDtypeStruct((_N, _GATD), jnp.float32),
            jax.ShapeDtypeStruct((_N, 1), jnp.float32),
            jax.ShapeDtypeStruct((_N, 1), jnp.float32),
        ],
    )(agg, x, root_W, root_b.reshape(1, -1), gat_W,
      att_src.reshape(1, -1), att_dst.reshape(1, -1))
    a_s = a_s[:, 0]
    a_d = a_d[:, 0]

    ar = jnp.arange(n, dtype=src.dtype)
    src2 = jnp.concatenate([src, ar])
    dst2 = jnp.concatenate([dst, ar])
    logit = jax.nn.leaky_relu(a_s[src2] + a_d[dst2], 0.2)
    m = jax.ops.segment_max(logit, dst2, num_segments=n)
    alpha = jnp.exp(logit - m[dst2])
    denom = jax.ops.segment_sum(alpha, dst2, num_segments=n)
    alpha = alpha / denom[dst2]
    out = jax.ops.segment_sum(alpha[:, None] * h[src2], dst2, num_segments=n)
    x2 = jnp.maximum(out + gat_b, 0.0)

    G = 16
    sums = jax.ops.segment_sum(x2, batch, num_segments=G)
    counts = jax.ops.segment_sum(jnp.ones((n,), x2.dtype), batch, num_segments=G)
    pooled = sums / jnp.maximum(counts, 1.0)[:, None]
    return jax.nn.relu(pooled @ fc_W + fc_b)
